# baseline (speedup 1.0000x reference)
import functools

import jax
import jax.numpy as jnp
import numpy as np
from jax.experimental import pallas as pl
from jax.experimental.pallas import tpu as pltpu

T, H = 8, 8
D_IN, D_OUT = 1024, 4096
S = 2048
BO = 1024
CS = 256


def _fused_body(x_ref, w_ref, b_ref, p_ref, pk_ref, pkt_ref, o_ref,
                xbf_ref, xc_ref):
    f32 = jnp.float32
    bf16 = jnp.bfloat16
    hi = jax.lax.Precision.HIGHEST
    pbf = p_ref[...].astype(bf16)

    @pl.when(pl.program_id(0) == 0)
    def _():
        for c in range(S // CS):
            xc = x_ref[pl.ds(c * CS, CS), :].astype(bf16)
            xbf_ref[pl.ds(c * CS, CS), :] = xc
            s = jax.lax.dot_general(xc, pbf, (((1,), (1,)), ((), ())),
                                    preferred_element_type=f32)
            bits = (s > 0.0).astype(f32)
            codes = jax.lax.dot_general(bits, pk_ref[...],
                                        (((1,), (0,)), ((), ())),
                                        precision=hi,
                                        preferred_element_type=f32)
            xc_ref[pl.ds(c * CS, CS), :] = codes.astype(bf16)

    wbf = w_ref[...].astype(bf16)
    sw = jax.lax.dot_general(wbf, pbf, (((1,), (1,)), ((), ())),
                             preferred_element_type=f32)
    wbits = (sw > 0.0).astype(f32)
    wct = jax.lax.dot_general(pkt_ref[...], wbits, (((1,), (1,)), ((), ())),
                              precision=hi,
                              preferred_element_type=f32).astype(bf16)

    bias = b_ref[...]
    for c in range(S // CS):
        xc = xbf_ref[pl.ds(c * CS, CS), :]
        dense = jax.lax.dot_general(xc, wbf, (((1,), (1,)), ((), ())),
                                    preferred_element_type=f32)
        dense = dense + bias
        codes = xc_ref[pl.ds(c * CS, CS), :]
        m = codes[:, 0:1] == wct[0:1, :]
        for t in range(1, T):
            m = jnp.logical_or(m, codes[:, t:t + 1] == wct[t:t + 1, :])
        o_ref[pl.ds(c * CS, CS), :] = jnp.where(m, dense, 0.0)


@functools.partial(jax.jit, static_argnames=())
def kernel(x, W, b, proj):
    B = x.shape[0]
    x2 = x.reshape(B * S, D_IN)
    proj2 = proj.reshape(T * H, D_IN)
    b2 = b.reshape(1, D_OUT)
    pk = np.zeros((T * H, T), dtype=np.float32)
    for t in range(T):
        for h in range(H):
            pk[t * H + h, t] = float(2 ** h)
    pkt = jnp.asarray(pk.T.copy())
    pk = jnp.asarray(pk)

    out = pl.pallas_call(
        _fused_body,
        grid=(D_OUT // BO,),
        in_specs=[
            pl.BlockSpec((B * S, D_IN), lambda o: (0, 0)),
            pl.BlockSpec((BO, D_IN), lambda o: (o, 0)),
            pl.BlockSpec((1, BO), lambda o: (0, o)),
            pl.BlockSpec((T * H, D_IN), lambda o: (0, 0)),
            pl.BlockSpec((T * H, T), lambda o: (0, 0)),
            pl.BlockSpec((T, T * H), lambda o: (0, 0)),
        ],
        out_specs=pl.BlockSpec((B * S, BO), lambda o: (0, o)),
        out_shape=jax.ShapeDtypeStruct((B * S, D_OUT), jnp.float32),
        scratch_shapes=[pltpu.VMEM((B * S, D_IN), jnp.bfloat16),
                        pltpu.VMEM((B * S, T), jnp.bfloat16)],
    )(x2, W, b2, proj2, pk, pkt)
    return out.reshape(B, S, D_OUT)


# speedup vs baseline: 1.0368x; 1.0368x over previous
import functools

import jax
import jax.numpy as jnp
import numpy as np
from jax.experimental import pallas as pl
from jax.experimental.pallas import tpu as pltpu

T, H = 8, 8
D_IN, D_OUT = 1024, 4096
S = 2048
BO = 512
CS = 256


def _fused_body(x_ref, w_ref, b_ref, p_ref, pk_ref, pkt_ref, o_ref,
                xbf_ref, xc_ref):
    f32 = jnp.float32
    bf16 = jnp.bfloat16
    hi = jax.lax.Precision.HIGHEST
    pbf = p_ref[...].astype(bf16)

    @pl.when(pl.program_id(0) == 0)
    def _():
        for c in range(S // CS):
            xc = x_ref[pl.ds(c * CS, CS), :].astype(bf16)
            xbf_ref[pl.ds(c * CS, CS), :] = xc
            s = jax.lax.dot_general(xc, pbf, (((1,), (1,)), ((), ())),
                                    preferred_element_type=f32)
            bits = (s > 0.0).astype(f32)
            codes = jax.lax.dot_general(bits, pk_ref[...],
                                        (((1,), (0,)), ((), ())),
                                        precision=hi,
                                        preferred_element_type=f32)
            xc_ref[pl.ds(c * CS, CS), :] = codes.astype(bf16)

    wbf = w_ref[...].astype(bf16)
    sw = jax.lax.dot_general(wbf, pbf, (((1,), (1,)), ((), ())),
                             preferred_element_type=f32)
    wbits = (sw > 0.0).astype(f32)
    wct = jax.lax.dot_general(pkt_ref[...], wbits, (((1,), (1,)), ((), ())),
                              precision=hi,
                              preferred_element_type=f32).astype(bf16)

    bias = b_ref[...]
    nc = S // CS

    def _dense(c):
        xc = xbf_ref[pl.ds(c * CS, CS), :]
        d = jax.lax.dot_general(xc, wbf, (((1,), (1,)), ((), ())),
                                preferred_element_type=f32)
        return d + bias

    dense_p = _dense(0)
    for c in range(nc):
        dense_n = _dense(c + 1) if c + 1 < nc else None
        codes = xc_ref[pl.ds(c * CS, CS), :]
        m = codes[:, 0:1] == wct[0:1, :]
        for t in range(1, T):
            m = jnp.logical_or(m, codes[:, t:t + 1] == wct[t:t + 1, :])
        o_ref[pl.ds(c * CS, CS), :] = jnp.where(m, dense_p, 0.0)
        dense_p = dense_n


@functools.partial(jax.jit, static_argnames=())
def kernel(x, W, b, proj):
    B = x.shape[0]
    x2 = x.reshape(B * S, D_IN)
    proj2 = proj.reshape(T * H, D_IN)
    b2 = b.reshape(1, D_OUT)
    pk = np.zeros((T * H, T), dtype=np.float32)
    for t in range(T):
        for h in range(H):
            pk[t * H + h, t] = float(2 ** h)
    pkt = jnp.asarray(pk.T.copy())
    pk = jnp.asarray(pk)

    out = pl.pallas_call(
        _fused_body,
        grid=(D_OUT // BO,),
        in_specs=[
            pl.BlockSpec((B * S, D_IN), lambda o: (0, 0)),
            pl.BlockSpec((BO, D_IN), lambda o: (o, 0)),
            pl.BlockSpec((1, BO), lambda o: (0, o)),
            pl.BlockSpec((T * H, D_IN), lambda o: (0, 0)),
            pl.BlockSpec((T * H, T), lambda o: (0, 0)),
            pl.BlockSpec((T, T * H), lambda o: (0, 0)),
        ],
        out_specs=pl.BlockSpec((B * S, BO), lambda o: (0, o)),
        out_shape=jax.ShapeDtypeStruct((B * S, D_OUT), jnp.float32),
        scratch_shapes=[pltpu.VMEM((B * S, D_IN), jnp.bfloat16),
                        pltpu.VMEM((B * S, T), jnp.bfloat16)],
    )(x2, W, b2, proj2, pk, pkt)
    return out.reshape(B, S, D_OUT)
